# baseline (speedup 1.0000x reference)
import jax
import jax.numpy as jnp
from jax.experimental import pallas as pl
from jax.experimental.pallas import tpu as pltpu

_SIZE = 16384
_INPUT = 4096
_K = 25
_NS = 2
_BR = 512
_NB = _SIZE // (_NS * _BR)


def _body(t_ref, *refs):
    perm_refs = refs[:_NS]
    out_ref = refs[_NS]
    ov_ref = refs[_NS + 1]
    i = pl.program_id(0)
    t = t_ref[...]
    for s in range(_NS):
        blk = perm_refs[s][0]
        mask = (blk > t).astype(jnp.float32)
        ov_ref[s * _NB + i, :] = jnp.sum(mask, axis=1)

    @pl.when(i == _NB - 1)
    def _finish():
        ovs = ov_ref[...]

        def step(_, carry):
            lo, hi = carry
            mid = (lo + hi) // 2
            cnt = jnp.sum((ovs >= mid.astype(jnp.float32)).astype(jnp.int32))
            ok = cnt >= _K + 1
            return jnp.where(ok, mid, lo), jnp.where(ok, hi, mid)

        lo, _ = jax.lax.fori_loop(
            0, 13, step, (jnp.int32(0), jnp.int32(_INPUT + 1)))
        out_ref[...] = (ovs > lo.astype(jnp.float32)).astype(jnp.int32)


def kernel(input_vector, perms):
    thresholds = jnp.where(input_vector, jnp.float32(0.6), jnp.inf)
    thresholds = thresholds.reshape(1, _INPUT)
    p3 = perms.reshape(_NS, _NB * _BR, _INPUT)
    out = pl.pallas_call(
        _body,
        grid=(_NB,),
        in_specs=[pl.BlockSpec((1, _INPUT), lambda i: (0, 0))] + [
            pl.BlockSpec((1, _BR, _INPUT), lambda i, s=s: (s, i, 0))
            for s in range(_NS)
        ],
        out_specs=pl.BlockSpec((_NS * _NB, _BR), lambda i: (0, 0)),
        out_shape=jax.ShapeDtypeStruct((_NS * _NB, _BR), jnp.int32),
        scratch_shapes=[pltpu.VMEM((_NS * _NB, _BR), jnp.float32)],
    )(thresholds, *([p3] * _NS))
    return out.reshape(_SIZE).astype(jnp.bool_)


# speedup vs baseline: 1.0381x; 1.0381x over previous
import jax
import jax.numpy as jnp
from jax.experimental import pallas as pl
from jax.experimental.pallas import tpu as pltpu

_SIZE = 16384
_INPUT = 4096
_K = 25
_NS = 1
_BR = 1024
_NB = _SIZE // (_NS * _BR)


def _body(t_ref, *refs):
    perm_refs = refs[:_NS]
    out_ref = refs[_NS]
    ov_ref = refs[_NS + 1]
    i = pl.program_id(0)
    t = t_ref[...]
    for s in range(_NS):
        blk = perm_refs[s][0]
        mask = (blk > t).astype(jnp.float32)
        ov_ref[s * _NB + i, :] = jnp.sum(mask, axis=1)

    @pl.when(i == _NB - 1)
    def _finish():
        ovs = ov_ref[...]

        def step(_, carry):
            lo, hi = carry
            mid = (lo + hi) // 2
            cnt = jnp.sum((ovs >= mid.astype(jnp.float32)).astype(jnp.int32))
            ok = cnt >= _K + 1
            return jnp.where(ok, mid, lo), jnp.where(ok, hi, mid)

        lo, _ = jax.lax.fori_loop(
            0, 13, step, (jnp.int32(0), jnp.int32(_INPUT + 1)))
        out_ref[...] = (ovs > lo.astype(jnp.float32)).astype(jnp.int32)


def kernel(input_vector, perms):
    thresholds = jnp.where(input_vector, jnp.float32(0.6), jnp.inf)
    thresholds = thresholds.reshape(1, _INPUT)
    p3 = perms.reshape(_NS, _NB * _BR, _INPUT)
    out = pl.pallas_call(
        _body,
        grid=(_NB,),
        in_specs=[pl.BlockSpec((1, _INPUT), lambda i: (0, 0))] + [
            pl.BlockSpec((1, _BR, _INPUT), lambda i, s=s: (s, i, 0))
            for s in range(_NS)
        ],
        out_specs=pl.BlockSpec((_NS * _NB, _BR), lambda i: (0, 0)),
        out_shape=jax.ShapeDtypeStruct((_NS * _NB, _BR), jnp.int32),
        scratch_shapes=[pltpu.VMEM((_NS * _NB, _BR), jnp.float32)],
    )(thresholds, *([p3] * _NS))
    return out.reshape(_SIZE).astype(jnp.bool_)
